# baseline (speedup 1.0000x reference)
import math

import jax
import jax.numpy as jnp
from jax.experimental import pallas as pl
from jax.experimental.pallas import tpu as pltpu


def _mm(x, W, b):
    M, K = x.shape
    N = W.shape[1]
    bm = 512
    Mp = ((M + bm - 1) // bm) * bm
    xp = jnp.pad(x, ((0, Mp - M), (0, 0)))

    def body(xr, wr, br, or_):
        or_[...] = (
            jnp.dot(xr[...], wr[...], preferred_element_type=jnp.float32)
            + br[...]
        )

    out = pl.pallas_call(
        body,
        grid=(Mp // bm,),
        in_specs=[
            pl.BlockSpec((bm, K), lambda i: (i, 0)),
            pl.BlockSpec((K, N), lambda i: (0, 0)),
            pl.BlockSpec((1, N), lambda i: (0, 0)),
        ],
        out_specs=pl.BlockSpec((bm, N), lambda i: (i, 0)),
        out_shape=jax.ShapeDtypeStruct((Mp, N), jnp.float32),
    )(xp, W, b.reshape(1, N))
    return out[:M]


def _gcn(x, src, dst, em, W, b):
    N = x.shape[0]
    deg = jnp.zeros((N,), x.dtype).at[dst].add(em)
    dis = jnp.where(deg > 0, jax.lax.rsqrt(jnp.where(deg > 0, deg, 1.0)), 0.0)
    norm = dis[src] * dis[dst] * em
    h = _mm(x, W, b * 0.0)
    out = jnp.zeros((N, W.shape[1]), x.dtype).at[dst].add(h[src] * norm[:, None])
    return out + b


def _bn(x, g, b):
    m = jnp.mean(x, axis=0)
    v = jnp.var(x, axis=0)
    return (x - m) * jax.lax.rsqrt(v + 1e-5) * g + b


def _sag(x, src, dst, em, batch, ratio, Wr, Ws, b):
    N = x.shape[0]
    agg = jnp.zeros((N, x.shape[1]), x.dtype).at[dst].add(x[src] * em[:, None])
    score = (agg @ Wr + b + x @ Ws).reshape(-1)
    k = int(math.ceil(ratio * N))
    perm = jnp.argsort(-score)[:k]
    xn = x[perm] * jnp.tanh(score[perm])[:, None]
    nmap = jnp.full((N,), -1, dtype=jnp.int32).at[perm].set(
        jnp.arange(k, dtype=jnp.int32))
    ns = nmap[src]
    nd = nmap[dst]
    valid = (ns >= 0) & (nd >= 0) & (em > 0)
    ns = jnp.where(valid, ns, 0)
    nd = jnp.where(valid, nd, 0)
    return xn, ns, nd, valid.astype(x.dtype), batch[perm]


def kernel(x, edge_index, batch, W1, b1, W2, b2, W3, b3, W4, b4,
           p1_Wr, p1_Ws, p1_b, p2_Wr, p2_Ws, p2_b, p3_Wr, p3_Ws, p3_b,
           g1, be1, g2, be2, g3, be3, linW, linb, lin2W, lin2b):
    src = edge_index[0].astype(jnp.int32)
    dst = edge_index[1].astype(jnp.int32)
    batch = batch.astype(jnp.int32)
    em = jnp.ones((src.shape[0],), x.dtype)
    h = jax.nn.relu(_bn(_gcn(x, src, dst, em, W1, b1), g1, be1))
    h, src, dst, em, batch = _sag(h, src, dst, em, batch, 0.6, p1_Wr, p1_Ws, p1_b)
    h = jax.nn.relu(_bn(_gcn(h, src, dst, em, W2, b2), g2, be2))
    h, src, dst, em, batch = _sag(h, src, dst, em, batch, 0.6, p2_Wr, p2_Ws, p2_b)
    h = jax.nn.relu(_bn(_gcn(h, src, dst, em, W3, b3), g3, be3))
    h, src, dst, em, batch = _sag(h, src, dst, em, batch, 0.5, p3_Wr, p3_Ws, p3_b)
    h = _gcn(h, src, dst, em, W4, b4)
    gmp = jax.ops.segment_max(h, batch, num_segments=1)
    wsi = jnp.stack([gmp], axis=0)
    x1 = jnp.mean(wsi, axis=0)
    out = jax.nn.relu(x1 @ linW + linb) @ lin2W + lin2b
    return out, x1


# speedup vs baseline: 1.2674x; 1.2674x over previous
import functools
import math

import jax
import jax.numpy as jnp
from jax import lax
from jax.experimental import pallas as pl
from jax.experimental.pallas import tpu as pltpu
from jax.experimental.pallas import tpu_sc as plsc

N_NODES = 10000
N_EDGES = 150000
CHUNK = 32
NW = 32


def _mm(x, W):
    M, K = x.shape
    N = W.shape[1]
    bm = 512
    Mp = ((M + bm - 1) // bm) * bm
    xp = jnp.pad(x, ((0, Mp - M), (0, 0)))

    def body(xr, wr, or_):
        or_[...] = jnp.dot(xr[...], wr[...], preferred_element_type=jnp.float32)

    out = pl.pallas_call(
        body,
        grid=(Mp // bm,),
        in_specs=[
            pl.BlockSpec((bm, K), lambda i: (i, 0)),
            pl.BlockSpec((K, N), lambda i: (0, 0)),
        ],
        out_specs=pl.BlockSpec((bm, N), lambda i: (i, 0)),
        out_shape=jax.ShapeDtypeStruct((Mp, N), jnp.float32),
    )(xp, W)
    return out[:M]


@functools.lru_cache(maxsize=None)
def _make_agg(F, EC, N):
    mesh = plsc.VectorSubcoreMesh(core_axis_name="c", subcore_axis_name="s")
    NJ = F // 16
    NCH = EC // CHUNK
    NOUT = N + 2 * NW

    @functools.partial(
        pl.kernel,
        out_type=jax.ShapeDtypeStruct((NOUT, F), jnp.float32),
        mesh=mesh,
        scratch_types=[
            pltpu.VMEM((CHUNK,), jnp.int32),
            pltpu.VMEM((CHUNK, F), jnp.float32),
            pltpu.VMEM((F,), jnp.float32),
            pltpu.VMEM((CHUNK + 16,), jnp.int32),
            pltpu.VMEM((CHUNK + 16,), jnp.float32),
            pltpu.VMEM((16,), jnp.int32),
            pltpu.SMEM((2,), jnp.int32),
            pltpu.SemaphoreType.DMA,
        ],
    )
    def kern(tbl_hbm, ns_hbm, dst_hbm, w_hbm, d0_hbm, out_hbm,
             gidx, rows, acc, dsts, wts, d0s, st, sem):
        wid = lax.axis_index("s") * 2 + lax.axis_index("c")
        e0 = pl.multiple_of(wid * EC, CHUNK)
        zv = jnp.zeros((16,), jnp.float32)

        def zero_acc():
            for j in range(NJ):
                acc[pl.ds(j * 16, 16)] = zv

        zero_acc()
        pltpu.async_copy(acc, out_hbm.at[N + 2 * wid], sem).wait()
        pltpu.async_copy(acc, out_hbm.at[N + 2 * wid + 1], sem).wait()

        pltpu.async_copy(d0_hbm.at[pl.ds(wid * 8, 8)],
                         d0s.at[pl.ds(0, 8)], sem).wait()
        st[0] = d0s[pl.ds(0, 16)][0]
        st[1] = 1

        def flush(dest_row):
            pltpu.async_copy(acc, out_hbm.at[dest_row], sem).wait()
            zero_acc()

        @pl.loop(0, NCH)
        def _(ci):
            eb = pl.multiple_of(e0 + ci * CHUNK, CHUNK)
            pltpu.async_copy(ns_hbm.at[pl.ds(eb, CHUNK)], gidx, sem).wait()
            pltpu.async_copy(dst_hbm.at[pl.ds(eb, CHUNK)],
                             dsts.at[pl.ds(0, CHUNK)], sem).wait()
            pltpu.async_copy(w_hbm.at[pl.ds(eb, CHUNK)],
                             wts.at[pl.ds(0, CHUNK)], sem).wait()
            pltpu.async_copy(tbl_hbm.at[gidx], rows, sem).wait()

            @pl.loop(0, CHUNK)
            def _(i):
                d = dsts[pl.ds(i, 16)][0]

                @pl.when(d != st[0])
                def _():
                    od = st[0]

                    @pl.when(st[1] == 1)
                    def _():
                        flush(N + 2 * wid)

                    @pl.when(st[1] == 0)
                    def _():
                        flush(od)

                    st[0] = d
                    st[1] = 0

                nrm = wts[pl.ds(i, 16)][0]
                for j in range(NJ):
                    sl = pl.ds(j * 16, 16)
                    acc[sl] = acc[sl] + rows[i, sl] * nrm

        @pl.when(st[1] == 1)
        def _():
            flush(N + 2 * wid)

        @pl.when(st[1] == 0)
        def _():
            flush(N + 2 * wid + 1)

    return kern


def _bn_masked(gout, sel_f, k, g, be):
    xm = gout * sel_f[:, None]
    m = jnp.sum(xm, axis=0) / k
    v = jnp.sum((xm - m * sel_f[:, None]) ** 2, axis=0) / k
    return (gout - m) * jax.lax.rsqrt(v + 1e-5) * g + be


def _topk_mask(score, k):
    u = jax.lax.bitcast_convert_type(score, jnp.uint32)
    m = jnp.where(u >> 31 != 0, jnp.uint32(0xFFFFFFFF), jnp.uint32(0x80000000))
    key = u ^ m

    def bit_step(i, t):
        bit = jnp.uint32(31) - i.astype(jnp.uint32)
        cand = t | (jnp.uint32(1) << bit)
        return jnp.where(jnp.sum((key >= cand).astype(jnp.int32)) >= k, cand, t)

    thr = jax.lax.fori_loop(0, 32, bit_step, jnp.uint32(0))
    gt = key > thr
    eq = key == thr
    n_gt = jnp.sum(gt.astype(jnp.int32))
    return gt | (eq & (jnp.cumsum(eq.astype(jnp.int32)) <= (k - n_gt)))


def kernel(x, edge_index, batch, W1, b1, W2, b2, W3, b3, W4, b4,
           p1_Wr, p1_Ws, p1_b, p2_Wr, p2_Ws, p2_b, p3_Wr, p3_Ws, p3_b,
           g1, be1, g2, be2, g3, be3, linW, linb, lin2W, lin2b):
    src = edge_index[0].astype(jnp.int32)
    dst = edge_index[1].astype(jnp.int32)
    N = x.shape[0]
    E = src.shape[0]

    ET = E + N
    EP = ((ET + NW * CHUNK - 1) // (NW * CHUNK)) * (NW * CHUNK)
    EC = EP // NW
    dst_all = jnp.concatenate([
        dst, jnp.arange(N, dtype=jnp.int32),
        jnp.full((EP - ET,), N - 1, jnp.int32)])
    src_all = jnp.concatenate([src, jnp.zeros((EP - E,), jnp.int32)])
    perm = jnp.argsort(dst_all, stable=True)
    dst_s = dst_all[perm]
    ns_s = src_all[perm]

    cut = jnp.arange(NW, dtype=jnp.int32) * EC
    dfirsts = dst_s[cut]
    dlasts = dst_s[cut + (EC - 1)]
    d0_tbl = jnp.tile(dfirsts[:, None], (1, 8)).reshape(-1)
    part_ids = jnp.stack([dfirsts, dlasts], axis=1).reshape(-1)

    def sc_aggregate(tbl, w_real):
        F = tbl.shape[1]
        w_s = jnp.concatenate([w_real, jnp.zeros((EP - E,), jnp.float32)])[perm]
        kern = _make_agg(F, EC, N)
        out = kern(tbl, ns_s, dst_s, w_s, d0_tbl)
        parts = out[N:]
        out = out[:N]
        out = out.at[part_ids].set(0.0)
        return out.at[part_ids].add(parts)

    em = jnp.ones((E,), jnp.float32)
    sel_f = jnp.ones((N,), jnp.float32)
    k_cur = N
    h = x
    params = [
        (W1, b1, g1, be1, p1_Wr, p1_Ws, p1_b, 0.6),
        (W2, b2, g2, be2, p2_Wr, p2_Ws, p2_b, 0.6),
        (W3, b3, g3, be3, p3_Wr, p3_Ws, p3_b, 0.5),
    ]

    def gcn(h_in, em_l, W, b):
        deg = jnp.zeros((N,), jnp.float32).at[dst].add(em_l)
        dis = jnp.where(deg > 0,
                        jax.lax.rsqrt(jnp.where(deg > 0, deg, 1.0)), 0.0)
        norm = dis[src] * dis[dst] * em_l
        hw = _mm(h_in, W)
        return sc_aggregate(hw, norm) + b

    for (W, b, g, be, Wr, Ws, pb, ratio) in params:
        gout = gcn(h, em, W, b)
        h = jax.nn.relu(_bn_masked(gout, sel_f, k_cur, g, be)) * sel_f[:, None]
        agg = sc_aggregate(h, em)
        F = h.shape[1]
        proj = jnp.zeros((F, 128), jnp.float32)
        proj = proj.at[:, 0].set(Wr[:, 0]).at[:, 1].set(Ws[:, 0])
        sc_a = _mm(agg, proj)[:, 0]
        sc_h = _mm(h, proj)[:, 1]
        score = sc_a + sc_h + pb[0]
        score = jnp.where(sel_f > 0, score, -jnp.inf)
        k_new = int(math.ceil(ratio * k_cur))
        sel = _topk_mask(score, k_new)
        tfac = jnp.where(sel, jnp.tanh(score), 0.0)
        h = h * tfac[:, None]
        sel_f = sel.astype(jnp.float32)
        em = em * sel_f[src] * sel_f[dst]
        k_cur = k_new

    h = gcn(h, em, W4, b4)
    neg = jnp.finfo(jnp.float32).min
    x1 = jnp.max(jnp.where(sel_f[:, None] > 0, h, neg), axis=0, keepdims=True)
    out = jax.nn.relu(x1 @ linW + linb) @ lin2W + lin2b
    return out, x1


# speedup vs baseline: 1.4248x; 1.1241x over previous
import functools
import math

import jax
import jax.numpy as jnp
from jax import lax
from jax.experimental import pallas as pl
from jax.experimental.pallas import tpu as pltpu
from jax.experimental.pallas import tpu_sc as plsc

N_NODES = 10000
N_EDGES = 150000
CHUNK = 32
NW = 32


def _mm(x, W):
    M, K = x.shape
    N = W.shape[1]
    bm = 512
    Mp = ((M + bm - 1) // bm) * bm
    xp = jnp.pad(x, ((0, Mp - M), (0, 0)))

    def body(xr, wr, or_):
        or_[...] = jnp.dot(xr[...], wr[...], preferred_element_type=jnp.float32)

    out = pl.pallas_call(
        body,
        grid=(Mp // bm,),
        in_specs=[
            pl.BlockSpec((bm, K), lambda i: (i, 0)),
            pl.BlockSpec((K, N), lambda i: (0, 0)),
        ],
        out_specs=pl.BlockSpec((bm, N), lambda i: (i, 0)),
        out_shape=jax.ShapeDtypeStruct((Mp, N), jnp.float32),
    )(xp, W)
    return out[:M]


@functools.lru_cache(maxsize=None)
def _make_agg(F, EC, N):
    mesh = plsc.VectorSubcoreMesh(core_axis_name="c", subcore_axis_name="s")
    NJ = F // 16
    NCH = EC // CHUNK
    NOUT = N + 2 * NW

    @functools.partial(
        pl.kernel,
        out_type=jax.ShapeDtypeStruct((NOUT, F), jnp.float32),
        mesh=mesh,
        scratch_types=[
            pltpu.VMEM((EC,), jnp.int32),
            pltpu.VMEM((2, CHUNK, F), jnp.float32),
            pltpu.VMEM((2, F), jnp.float32),
            pltpu.VMEM((EC + 16,), jnp.int32),
            pltpu.VMEM((EC + 16,), jnp.float32),
            pltpu.VMEM((16,), jnp.int32),
            pltpu.SMEM((8,), jnp.int32),
            pltpu.SemaphoreType.DMA,
            pltpu.SemaphoreType.DMA,
            pltpu.SemaphoreType.DMA,
            pltpu.SemaphoreType.DMA,
            pltpu.SemaphoreType.DMA,
        ],
    )
    def kern(tbl_hbm, ns_hbm, dst_hbm, w_hbm, d0_hbm, out_hbm,
             gidx, rows, acc, dsts, wts, d0s, st, sem, gs0, gs1, fs0, fs1):
        wid = lax.axis_index("s") * 2 + lax.axis_index("c")
        e0 = pl.multiple_of(wid * EC, CHUNK)
        zv = jnp.zeros((16,), jnp.float32)

        def zero_acc(s):
            for j in range(NJ):
                acc[s, pl.ds(j * 16, 16)] = zv

        zero_acc(0)
        zero_acc(1)
        pltpu.async_copy(acc.at[0], out_hbm.at[N + 2 * wid], sem).wait()
        pltpu.async_copy(acc.at[0], out_hbm.at[N + 2 * wid + 1], sem).wait()

        pltpu.async_copy(ns_hbm.at[pl.ds(e0, EC)], gidx, sem).wait()
        pltpu.async_copy(dst_hbm.at[pl.ds(e0, EC)],
                         dsts.at[pl.ds(0, EC)], sem).wait()
        pltpu.async_copy(w_hbm.at[pl.ds(e0, EC)],
                         wts.at[pl.ds(0, EC)], sem).wait()
        pltpu.async_copy(d0_hbm.at[pl.ds(wid * 8, 8)],
                         d0s.at[pl.ds(0, 8)], sem).wait()
        st[0] = d0s[pl.ds(0, 16)][0]
        st[1] = 1
        st[2] = 0
        st[3] = 0
        st[4] = 0

        def gather(ci, buf, gsem):
            pltpu.async_copy(
                tbl_hbm.at[gidx.at[pl.ds(ci * CHUNK, CHUNK)]],
                rows.at[buf], gsem)

        def gather_wait(ci, buf, gsem):
            pltpu.make_async_copy(
                tbl_hbm.at[gidx.at[pl.ds(ci * CHUNK, CHUNK)]],
                rows.at[buf], gsem).wait()

        def flush(dest_row):
            s = st[2]

            @pl.when(s == 0)
            def _():
                pltpu.async_copy(acc.at[0], out_hbm.at[dest_row], fs0)
                st[3] = 1

            @pl.when(s == 1)
            def _():
                pltpu.async_copy(acc.at[1], out_hbm.at[dest_row], fs1)
                st[4] = 1

            ns = 1 - s
            st[2] = ns

            @pl.when((ns == 0) & (st[3] == 1))
            def _():
                pltpu.make_async_copy(acc.at[0], out_hbm.at[0], fs0).wait()
                st[3] = 0

            @pl.when((ns == 1) & (st[4] == 1))
            def _():
                pltpu.make_async_copy(acc.at[1], out_hbm.at[0], fs1).wait()
                st[4] = 0

            zero_acc(ns)

        gather(0, 0, gs0)

        @pl.loop(0, NCH)
        def _(ci):
            cur = lax.rem(ci, 2)

            @pl.when(ci + 1 < NCH)
            def _():
                @pl.when(cur == 0)
                def _():
                    gather(ci + 1, 1, gs1)

                @pl.when(cur == 1)
                def _():
                    gather(ci + 1, 0, gs0)

            @pl.when(cur == 0)
            def _():
                gather_wait(ci, 0, gs0)

            @pl.when(cur == 1)
            def _():
                gather_wait(ci, 1, gs1)

            @pl.loop(0, CHUNK)
            def _(i):
                e = ci * CHUNK + i
                d = dsts[pl.ds(e, 16)][0]

                @pl.when(d != st[0])
                def _():
                    od = st[0]

                    @pl.when(st[1] == 1)
                    def _():
                        flush(N + 2 * wid)

                    @pl.when(st[1] == 0)
                    def _():
                        flush(od)

                    st[0] = d
                    st[1] = 0

                nrm = wts[pl.ds(e, 16)][0]
                sa = st[2]
                for j in range(NJ):
                    sl = pl.ds(j * 16, 16)
                    acc[sa, sl] = acc[sa, sl] + rows[cur, i, sl] * nrm

        @pl.when(st[1] == 1)
        def _():
            flush(N + 2 * wid)

        @pl.when(st[1] == 0)
        def _():
            flush(N + 2 * wid + 1)

        @pl.when(st[3] == 1)
        def _():
            pltpu.make_async_copy(acc.at[0], out_hbm.at[0], fs0).wait()

        @pl.when(st[4] == 1)
        def _():
            pltpu.make_async_copy(acc.at[1], out_hbm.at[0], fs1).wait()

    return kern


def _bn_masked(gout, sel_f, k, g, be):
    xm = gout * sel_f[:, None]
    m = jnp.sum(xm, axis=0) / k
    v = jnp.sum((xm - m * sel_f[:, None]) ** 2, axis=0) / k
    return (gout - m) * jax.lax.rsqrt(v + 1e-5) * g + be


def _topk_mask(score, k):
    u = jax.lax.bitcast_convert_type(score, jnp.uint32)
    m = jnp.where(u >> 31 != 0, jnp.uint32(0xFFFFFFFF), jnp.uint32(0x80000000))
    key = u ^ m

    def bit_step(i, t):
        bit = jnp.uint32(31) - i.astype(jnp.uint32)
        cand = t | (jnp.uint32(1) << bit)
        return jnp.where(jnp.sum((key >= cand).astype(jnp.int32)) >= k, cand, t)

    thr = jax.lax.fori_loop(0, 32, bit_step, jnp.uint32(0))
    gt = key > thr
    eq = key == thr
    n_gt = jnp.sum(gt.astype(jnp.int32))
    return gt | (eq & (jnp.cumsum(eq.astype(jnp.int32)) <= (k - n_gt)))


def kernel(x, edge_index, batch, W1, b1, W2, b2, W3, b3, W4, b4,
           p1_Wr, p1_Ws, p1_b, p2_Wr, p2_Ws, p2_b, p3_Wr, p3_Ws, p3_b,
           g1, be1, g2, be2, g3, be3, linW, linb, lin2W, lin2b):
    src = edge_index[0].astype(jnp.int32)
    dst = edge_index[1].astype(jnp.int32)
    N = x.shape[0]
    E = src.shape[0]

    ET = E + N
    EP = ((ET + NW * CHUNK - 1) // (NW * CHUNK)) * (NW * CHUNK)
    EC = EP // NW
    dst_all = jnp.concatenate([
        dst, jnp.arange(N, dtype=jnp.int32),
        jnp.full((EP - ET,), N - 1, jnp.int32)])
    src_all = jnp.concatenate([src, jnp.zeros((EP - E,), jnp.int32)])
    perm = jnp.argsort(dst_all, stable=True)
    dst_s = dst_all[perm]
    ns_s = src_all[perm]

    cut = jnp.arange(NW, dtype=jnp.int32) * EC
    dfirsts = dst_s[cut]
    dlasts = dst_s[cut + (EC - 1)]
    d0_tbl = jnp.tile(dfirsts[:, None], (1, 8)).reshape(-1)
    part_ids = jnp.stack([dfirsts, dlasts], axis=1).reshape(-1)

    def sc_aggregate(tbl, w_real):
        F = tbl.shape[1]
        w_s = jnp.concatenate([w_real, jnp.zeros((EP - E,), jnp.float32)])[perm]
        kern = _make_agg(F, EC, N)
        out = kern(tbl, ns_s, dst_s, w_s, d0_tbl)
        parts = out[N:]
        out = out[:N]
        out = out.at[part_ids].set(0.0)
        return out.at[part_ids].add(parts)

    em = jnp.ones((E,), jnp.float32)
    sel_f = jnp.ones((N,), jnp.float32)
    k_cur = N
    h = x
    params = [
        (W1, b1, g1, be1, p1_Wr, p1_Ws, p1_b, 0.6),
        (W2, b2, g2, be2, p2_Wr, p2_Ws, p2_b, 0.6),
        (W3, b3, g3, be3, p3_Wr, p3_Ws, p3_b, 0.5),
    ]

    def gcn(h_in, em_l, W, b):
        deg = jnp.zeros((N,), jnp.float32).at[dst].add(em_l)
        dis = jnp.where(deg > 0,
                        jax.lax.rsqrt(jnp.where(deg > 0, deg, 1.0)), 0.0)
        norm = dis[src] * dis[dst] * em_l
        hw = _mm(h_in, W)
        return sc_aggregate(hw, norm) + b

    for (W, b, g, be, Wr, Ws, pb, ratio) in params:
        gout = gcn(h, em, W, b)
        h = jax.nn.relu(_bn_masked(gout, sel_f, k_cur, g, be)) * sel_f[:, None]
        agg = sc_aggregate(h, em)
        F = h.shape[1]
        proj = jnp.zeros((F, 128), jnp.float32)
        proj = proj.at[:, 0].set(Wr[:, 0]).at[:, 1].set(Ws[:, 0])
        sc_a = _mm(agg, proj)[:, 0]
        sc_h = _mm(h, proj)[:, 1]
        score = sc_a + sc_h + pb[0]
        score = jnp.where(sel_f > 0, score, -jnp.inf)
        k_new = int(math.ceil(ratio * k_cur))
        sel = _topk_mask(score, k_new)
        tfac = jnp.where(sel, jnp.tanh(score), 0.0)
        h = h * tfac[:, None]
        sel_f = sel.astype(jnp.float32)
        em = em * sel_f[src] * sel_f[dst]
        k_cur = k_new

    h = gcn(h, em, W4, b4)
    neg = jnp.finfo(jnp.float32).min
    x1 = jnp.max(jnp.where(sel_f[:, None] > 0, h, neg), axis=0, keepdims=True)
    out = jax.nn.relu(x1 @ linW + linb) @ lin2W + lin2b
    return out, x1


# speedup vs baseline: 1.4258x; 1.0007x over previous
import functools
import math

import jax
import jax.numpy as jnp
from jax import lax
from jax.experimental import pallas as pl
from jax.experimental.pallas import tpu as pltpu
from jax.experimental.pallas import tpu_sc as plsc

N_NODES = 10000
N_EDGES = 150000
CHUNK = 32
NW = 32


def _mm(x, W):
    M, K = x.shape
    N = W.shape[1]
    bm = 512
    Mp = ((M + bm - 1) // bm) * bm
    xp = jnp.pad(x, ((0, Mp - M), (0, 0)))

    def body(xr, wr, or_):
        or_[...] = jnp.dot(xr[...], wr[...], preferred_element_type=jnp.float32)

    out = pl.pallas_call(
        body,
        grid=(Mp // bm,),
        in_specs=[
            pl.BlockSpec((bm, K), lambda i: (i, 0)),
            pl.BlockSpec((K, N), lambda i: (0, 0)),
        ],
        out_specs=pl.BlockSpec((bm, N), lambda i: (i, 0)),
        out_shape=jax.ShapeDtypeStruct((Mp, N), jnp.float32),
    )(xp, W)
    return out[:M]


@functools.lru_cache(maxsize=None)
def _make_agg(F, EC, N):
    mesh = plsc.VectorSubcoreMesh(core_axis_name="c", subcore_axis_name="s")
    NJ = F // 16
    NCH = EC // CHUNK
    NOUT = N + 2 * NW

    @functools.partial(
        pl.kernel,
        out_type=jax.ShapeDtypeStruct((NOUT, F), jnp.float32),
        mesh=mesh,
        scratch_types=[
            pltpu.VMEM((EC,), jnp.int32),
            pltpu.VMEM((2, CHUNK, F), jnp.float32),
            pltpu.VMEM((2, F), jnp.float32),
            pltpu.VMEM((EC + 16,), jnp.int32),
            pltpu.VMEM((EC + 16,), jnp.float32),
            pltpu.VMEM((16,), jnp.int32),
            pltpu.SMEM((8,), jnp.int32),
            pltpu.SemaphoreType.DMA,
            pltpu.SemaphoreType.DMA,
            pltpu.SemaphoreType.DMA,
            pltpu.SemaphoreType.DMA,
            pltpu.SemaphoreType.DMA,
        ],
    )
    def kern(tbl_hbm, ns_hbm, dst_hbm, w_hbm, d0_hbm, out_hbm,
             gidx, rows, acc, dsts, wts, d0s, st, sem, gs0, gs1, fs0, fs1):
        wid = lax.axis_index("s") * 2 + lax.axis_index("c")
        e0 = pl.multiple_of(wid * EC, CHUNK)
        zv = jnp.zeros((16,), jnp.float32)

        def zero_acc(s):
            for j in range(NJ):
                acc[s, pl.ds(j * 16, 16)] = zv

        zero_acc(0)
        zero_acc(1)
        pltpu.async_copy(acc.at[0], out_hbm.at[N + 2 * wid], sem).wait()
        pltpu.async_copy(acc.at[0], out_hbm.at[N + 2 * wid + 1], sem).wait()

        pltpu.async_copy(ns_hbm.at[pl.ds(e0, EC)], gidx, sem).wait()
        pltpu.async_copy(dst_hbm.at[pl.ds(e0, EC)],
                         dsts.at[pl.ds(0, EC)], sem).wait()
        pltpu.async_copy(w_hbm.at[pl.ds(e0, EC)],
                         wts.at[pl.ds(0, EC)], sem).wait()
        pltpu.async_copy(d0_hbm.at[pl.ds(wid * 8, 8)],
                         d0s.at[pl.ds(0, 8)], sem).wait()
        st[0] = d0s[pl.ds(0, 16)][0]
        st[1] = 1
        st[2] = 0
        st[3] = 0
        st[4] = 0

        def gather(ci, buf, gsem):
            pltpu.async_copy(
                tbl_hbm.at[gidx.at[pl.ds(ci * CHUNK, CHUNK)]],
                rows.at[buf], gsem)

        def gather_wait(ci, buf, gsem):
            pltpu.make_async_copy(
                tbl_hbm.at[gidx.at[pl.ds(ci * CHUNK, CHUNK)]],
                rows.at[buf], gsem).wait()

        def flush(dest_row):
            s = st[2]

            @pl.when(s == 0)
            def _():
                pltpu.async_copy(acc.at[0], out_hbm.at[dest_row], fs0)
                st[3] = 1

            @pl.when(s == 1)
            def _():
                pltpu.async_copy(acc.at[1], out_hbm.at[dest_row], fs1)
                st[4] = 1

            ns = 1 - s
            st[2] = ns

            @pl.when((ns == 0) & (st[3] == 1))
            def _():
                pltpu.make_async_copy(acc.at[0], out_hbm.at[0], fs0).wait()
                st[3] = 0

            @pl.when((ns == 1) & (st[4] == 1))
            def _():
                pltpu.make_async_copy(acc.at[1], out_hbm.at[0], fs1).wait()
                st[4] = 0

            zero_acc(ns)

        gather(0, 0, gs0)

        @pl.loop(0, NCH)
        def _(ci):
            cur = lax.rem(ci, 2)

            @pl.when(ci + 1 < NCH)
            def _():
                @pl.when(cur == 0)
                def _():
                    gather(ci + 1, 1, gs1)

                @pl.when(cur == 1)
                def _():
                    gather(ci + 1, 0, gs0)

            @pl.when(cur == 0)
            def _():
                gather_wait(ci, 0, gs0)

            @pl.when(cur == 1)
            def _():
                gather_wait(ci, 1, gs1)

            @pl.loop(0, CHUNK)
            def _(i):
                e = ci * CHUNK + i
                d = dsts[pl.ds(e, 16)][0]

                @pl.when(d != st[0])
                def _():
                    od = st[0]

                    @pl.when(st[1] == 1)
                    def _():
                        flush(N + 2 * wid)

                    @pl.when(st[1] == 0)
                    def _():
                        flush(od)

                    st[0] = d
                    st[1] = 0

                nrm = wts[pl.ds(e, 16)][0]
                sa = st[2]
                for j in range(NJ):
                    sl = pl.ds(j * 16, 16)
                    acc[sa, sl] = acc[sa, sl] + rows[cur, i, sl] * nrm

        @pl.when(st[1] == 1)
        def _():
            flush(N + 2 * wid)

        @pl.when(st[1] == 0)
        def _():
            flush(N + 2 * wid + 1)

        @pl.when(st[3] == 1)
        def _():
            pltpu.make_async_copy(acc.at[0], out_hbm.at[0], fs0).wait()

        @pl.when(st[4] == 1)
        def _():
            pltpu.make_async_copy(acc.at[1], out_hbm.at[0], fs1).wait()

    return kern


def _bn_masked(gout, sel_f, k, g, be):
    xm = gout * sel_f[:, None]
    m = jnp.sum(xm, axis=0) / k
    v = jnp.sum((xm - m * sel_f[:, None]) ** 2, axis=0) / k
    return (gout - m) * jax.lax.rsqrt(v + 1e-5) * g + be


def _topk_mask(score, k):
    u = jax.lax.bitcast_convert_type(score, jnp.uint32)
    m = jnp.where(u >> 31 != 0, jnp.uint32(0xFFFFFFFF), jnp.uint32(0x80000000))
    key = u ^ m

    def bit_step(i, t):
        bit = jnp.uint32(31) - i.astype(jnp.uint32)
        cand = t | (jnp.uint32(1) << bit)
        return jnp.where(jnp.sum((key >= cand).astype(jnp.int32)) >= k, cand, t)

    thr = jax.lax.fori_loop(0, 32, bit_step, jnp.uint32(0))
    gt = key > thr
    eq = key == thr
    n_gt = jnp.sum(gt.astype(jnp.int32))
    return gt | (eq & (jnp.cumsum(eq.astype(jnp.int32)) <= (k - n_gt)))


def kernel(x, edge_index, batch, W1, b1, W2, b2, W3, b3, W4, b4,
           p1_Wr, p1_Ws, p1_b, p2_Wr, p2_Ws, p2_b, p3_Wr, p3_Ws, p3_b,
           g1, be1, g2, be2, g3, be3, linW, linb, lin2W, lin2b):
    src = edge_index[0].astype(jnp.int32)
    dst = edge_index[1].astype(jnp.int32)
    N = x.shape[0]
    E = src.shape[0]

    ET = E + N
    EP = ((ET + NW * CHUNK - 1) // (NW * CHUNK)) * (NW * CHUNK)
    EC = EP // NW
    dst_all = jnp.concatenate([
        dst, jnp.arange(N, dtype=jnp.int32),
        jnp.full((EP - ET,), N - 1, jnp.int32)])
    src_all = jnp.concatenate([src, jnp.zeros((EP - E,), jnp.int32)])
    key = ((dst_all.astype(jnp.uint32) << 18)
           | jnp.arange(EP, dtype=jnp.uint32))
    key_s = jnp.sort(key)
    perm = (key_s & jnp.uint32(0x3FFFF)).astype(jnp.int32)
    dst_s = (key_s >> 18).astype(jnp.int32)
    ns_s = src_all[perm]

    cut = jnp.arange(NW, dtype=jnp.int32) * EC
    dfirsts = dst_s[cut]
    dlasts = dst_s[cut + (EC - 1)]
    d0_tbl = jnp.tile(dfirsts[:, None], (1, 8)).reshape(-1)
    part_ids = jnp.stack([dfirsts, dlasts], axis=1).reshape(-1)

    def sc_aggregate(tbl, w_real):
        F = tbl.shape[1]
        w_s = jnp.concatenate([w_real, jnp.zeros((EP - E,), jnp.float32)])[perm]
        kern = _make_agg(F, EC, N)
        out = kern(tbl, ns_s, dst_s, w_s, d0_tbl)
        parts = out[N:]
        out = out[:N]
        out = out.at[part_ids].set(0.0)
        return out.at[part_ids].add(parts)

    em = jnp.ones((E,), jnp.float32)
    sel_f = jnp.ones((N,), jnp.float32)
    k_cur = N
    h = x
    params = [
        (W1, b1, g1, be1, p1_Wr, p1_Ws, p1_b, 0.6),
        (W2, b2, g2, be2, p2_Wr, p2_Ws, p2_b, 0.6),
        (W3, b3, g3, be3, p3_Wr, p3_Ws, p3_b, 0.5),
    ]

    def gcn(h_in, em_l, W, b):
        deg = jnp.zeros((N,), jnp.float32).at[dst].add(em_l)
        dis = jnp.where(deg > 0,
                        jax.lax.rsqrt(jnp.where(deg > 0, deg, 1.0)), 0.0)
        norm = dis[src] * dis[dst] * em_l
        hw = _mm(h_in, W)
        return sc_aggregate(hw, norm) + b

    for (W, b, g, be, Wr, Ws, pb, ratio) in params:
        gout = gcn(h, em, W, b)
        h = jax.nn.relu(_bn_masked(gout, sel_f, k_cur, g, be)) * sel_f[:, None]
        agg = sc_aggregate(h, em)
        F = h.shape[1]
        proj = jnp.zeros((F, 128), jnp.float32)
        proj = proj.at[:, 0].set(Wr[:, 0]).at[:, 1].set(Ws[:, 0])
        sc_a = _mm(agg, proj)[:, 0]
        sc_h = _mm(h, proj)[:, 1]
        score = sc_a + sc_h + pb[0]
        score = jnp.where(sel_f > 0, score, -jnp.inf)
        k_new = int(math.ceil(ratio * k_cur))
        sel = _topk_mask(score, k_new)
        tfac = jnp.where(sel, jnp.tanh(score), 0.0)
        h = h * tfac[:, None]
        sel_f = sel.astype(jnp.float32)
        em = em * sel_f[src] * sel_f[dst]
        k_cur = k_new

    h = gcn(h, em, W4, b4)
    neg = jnp.finfo(jnp.float32).min
    x1 = jnp.max(jnp.where(sel_f[:, None] > 0, h, neg), axis=0, keepdims=True)
    out = jax.nn.relu(x1 @ linW + linb) @ lin2W + lin2b
    return out, x1
